# baseline (device time: 4252836 ns/iter reference)
import contextlib
import os

import jax
import jax.numpy as jnp
from jax import lax
from jax.experimental import pallas as pl
from jax.experimental.pallas import tpu as pltpu

_PROF = os.environ.get("KPROF") == "1"


def _scope(name):
    return jax.named_scope(name) if _PROF else contextlib.nullcontext()


K = 8


def kernel(x):
    m, n = x.shape
    mc = m // K

    def body(x_ref, out_ref, send_sems, recv_sems, copy_sem):
        my_x = lax.axis_index("x")
        my_y = lax.axis_index("y")
        my_z = lax.axis_index("z")
        peer = (my_x, 1 - my_y, my_z)

        with _scope("barrier"):
            barrier = pltpu.get_barrier_semaphore()
            pl.semaphore_signal(
                barrier, inc=1, device_id=peer, device_id_type=pl.DeviceIdType.MESH
            )
            pl.semaphore_wait(barrier, 1)

        with _scope("issue"):
            local = pltpu.make_async_copy(
                x_ref, out_ref.at[pl.ds(my_y * m, m), :], copy_sem
            )
            local.start()

            rdmas = []
            for k in range(K):
                rdma = pltpu.make_async_remote_copy(
                    src_ref=x_ref.at[pl.ds(k * mc, mc), :],
                    dst_ref=out_ref.at[pl.ds(my_y * m + k * mc, mc), :],
                    send_sem=send_sems.at[k],
                    recv_sem=recv_sems.at[k],
                    device_id=peer,
                    device_id_type=pl.DeviceIdType.MESH,
                )
                rdma.start()
                rdmas.append(rdma)

        with _scope("wait_local"):
            local.wait()
        with _scope("wait"):
            for rdma in rdmas:
                rdma.wait()

    return pl.pallas_call(
        body,
        out_shape=jax.ShapeDtypeStruct((2 * m, n), x.dtype),
        in_specs=[pl.BlockSpec(memory_space=pl.ANY)],
        out_specs=pl.BlockSpec(memory_space=pl.ANY),
        scratch_shapes=[
            pltpu.SemaphoreType.DMA((K,)),
            pltpu.SemaphoreType.DMA((K,)),
            pltpu.SemaphoreType.DMA,
        ],
        compiler_params=pltpu.CompilerParams(collective_id=0),
    )(x)


# device time: 944252 ns/iter; 4.5039x vs baseline; 4.5039x over previous
import functools

import jax
import jax.numpy as jnp
from jax import lax
from jax.experimental import pallas as pl
from jax.experimental.pallas import tpu as pltpu

K = 16
KC = 16


def kernel(x):
    m, n = x.shape
    h = m // 2
    rc = h // K
    lc = m // KC

    def body(
        x_ref,
        out_ref,
        ysend,
        yrecv,
        xsend,
        xrecv,
        vbuf,
        isems,
        osems,
    ):
        my_x = lax.axis_index("x")
        my_y = lax.axis_index("y")
        my_z = lax.axis_index("z")
        peer_y = (my_x, 1 - my_y, my_z)
        peer_x = (1 - my_x, my_y, my_z)

        barrier = pltpu.get_barrier_semaphore()
        for nbr in (peer_y, peer_x):
            pl.semaphore_signal(
                barrier, inc=1, device_id=nbr, device_id_type=pl.DeviceIdType.MESH
            )
        pl.semaphore_wait(barrier, 2)

        mine_out = my_y * m
        frn_out = (1 - my_y) * m
        half = my_x * h

        y_rdmas = []
        for k in range(K):
            off = half + k * rc
            r = pltpu.make_async_remote_copy(
                src_ref=x_ref.at[pl.ds(off, rc), :],
                dst_ref=out_ref.at[pl.ds(mine_out + off, rc), :],
                send_sem=ysend.at[k],
                recv_sem=yrecv.at[k],
                device_id=peer_y,
                device_id_type=pl.DeviceIdType.MESH,
            )
            r.start()
            y_rdmas.append(r)

        in_cps = [
            pltpu.make_async_copy(
                x_ref.at[pl.ds(k * lc, lc), :], vbuf.at[k % 2], isems.at[k % 2]
            )
            for k in range(KC)
        ]
        out_cps = [
            pltpu.make_async_copy(
                vbuf.at[k % 2],
                out_ref.at[pl.ds(mine_out + k * lc, lc), :],
                osems.at[k % 2],
            )
            for k in range(KC)
        ]

        def stage_step(k):
            if k >= KC:
                return
            if k == 0:
                in_cps[0].start()
            in_cps[k].wait()
            out_cps[k].start()
            if k + 1 < KC:
                if k >= 1:
                    out_cps[k - 1].wait()
                in_cps[k + 1].start()

        x_rdmas = []
        for k in range(K):
            y_rdmas[k].wait_recv()
            off = frn_out + half + k * rc
            r = pltpu.make_async_remote_copy(
                src_ref=out_ref.at[pl.ds(off, rc), :],
                dst_ref=out_ref.at[pl.ds(off, rc), :],
                send_sem=xsend.at[k],
                recv_sem=xrecv.at[k],
                device_id=peer_x,
                device_id_type=pl.DeviceIdType.MESH,
            )
            r.start()
            x_rdmas.append(r)
            stage_step(k)

        for k in range(K, KC):
            stage_step(k)
        for k in range(K):
            y_rdmas[k].wait_send()
            x_rdmas[k].wait_send()
            x_rdmas[k].wait_recv()
        if KC >= 2:
            out_cps[KC - 2].wait()
        out_cps[KC - 1].wait()

        @functools.partial(
            pl.run_scoped, second_barrier=pltpu.SemaphoreType.REGULAR
        )
        def _(second_barrier):
            for nbr in (peer_y, peer_x):
                pl.semaphore_signal(
                    second_barrier,
                    inc=1,
                    device_id=nbr,
                    device_id_type=pl.DeviceIdType.MESH,
                )
            pl.semaphore_wait(second_barrier, 2)

    return pl.pallas_call(
        body,
        out_shape=jax.ShapeDtypeStruct((2 * m, n), x.dtype),
        in_specs=[pl.BlockSpec(memory_space=pl.ANY)],
        out_specs=pl.BlockSpec(memory_space=pl.ANY),
        scratch_shapes=[
            pltpu.SemaphoreType.DMA((K,)),
            pltpu.SemaphoreType.DMA((K,)),
            pltpu.SemaphoreType.DMA((K,)),
            pltpu.SemaphoreType.DMA((K,)),
            pltpu.VMEM((2, m // KC, n), jnp.float32),
            pltpu.SemaphoreType.DMA((2,)),
            pltpu.SemaphoreType.DMA((2,)),
        ],
        compiler_params=pltpu.CompilerParams(collective_id=0),
    )(x)


# device time: 926183 ns/iter; 4.5918x vs baseline; 1.0195x over previous
import functools

import jax
import jax.numpy as jnp
from jax import lax
from jax.experimental import pallas as pl
from jax.experimental.pallas import tpu as pltpu

K = 32
KC = 16


def kernel(x):
    m, n = x.shape
    h = m // 2
    rc = h // K
    lc = m // KC

    def body(
        x_ref,
        out_ref,
        ysend,
        yrecv,
        xsend,
        xrecv,
        vbuf,
        isems,
        osems,
    ):
        my_x = lax.axis_index("x")
        my_y = lax.axis_index("y")
        my_z = lax.axis_index("z")
        peer_y = (my_x, 1 - my_y, my_z)
        peer_x = (1 - my_x, my_y, my_z)

        barrier = pltpu.get_barrier_semaphore()
        for nbr in (peer_y, peer_x):
            pl.semaphore_signal(
                barrier, inc=1, device_id=nbr, device_id_type=pl.DeviceIdType.MESH
            )
        pl.semaphore_wait(barrier, 2)

        mine_out = my_y * m
        frn_out = (1 - my_y) * m
        half = my_x * h

        y_rdmas = []
        for k in range(K):
            off = half + k * rc
            r = pltpu.make_async_remote_copy(
                src_ref=x_ref.at[pl.ds(off, rc), :],
                dst_ref=out_ref.at[pl.ds(mine_out + off, rc), :],
                send_sem=ysend.at[k],
                recv_sem=yrecv.at[k],
                device_id=peer_y,
                device_id_type=pl.DeviceIdType.MESH,
            )
            r.start()
            y_rdmas.append(r)

        in_cps = [
            pltpu.make_async_copy(
                x_ref.at[pl.ds(k * lc, lc), :], vbuf.at[k % 2], isems.at[k % 2]
            )
            for k in range(KC)
        ]
        out_cps = [
            pltpu.make_async_copy(
                vbuf.at[k % 2],
                out_ref.at[pl.ds(mine_out + k * lc, lc), :],
                osems.at[k % 2],
            )
            for k in range(KC)
        ]

        def stage_step(k):
            if k >= KC:
                return
            if k == 0:
                in_cps[0].start()
            in_cps[k].wait()
            out_cps[k].start()
            if k + 1 < KC:
                if k >= 1:
                    out_cps[k - 1].wait()
                in_cps[k + 1].start()

        x_rdmas = []
        for k in range(K):
            y_rdmas[k].wait_recv()
            off = frn_out + half + k * rc
            r = pltpu.make_async_remote_copy(
                src_ref=out_ref.at[pl.ds(off, rc), :],
                dst_ref=out_ref.at[pl.ds(off, rc), :],
                send_sem=xsend.at[k],
                recv_sem=xrecv.at[k],
                device_id=peer_x,
                device_id_type=pl.DeviceIdType.MESH,
            )
            r.start()
            x_rdmas.append(r)
            stage_step(k)

        for k in range(K, KC):
            stage_step(k)
        for k in range(K):
            y_rdmas[k].wait_send()
            x_rdmas[k].wait_send()
            x_rdmas[k].wait_recv()
        if KC >= 2:
            out_cps[KC - 2].wait()
        out_cps[KC - 1].wait()

        @functools.partial(
            pl.run_scoped, second_barrier=pltpu.SemaphoreType.REGULAR
        )
        def _(second_barrier):
            for nbr in (peer_y, peer_x):
                pl.semaphore_signal(
                    second_barrier,
                    inc=1,
                    device_id=nbr,
                    device_id_type=pl.DeviceIdType.MESH,
                )
            pl.semaphore_wait(second_barrier, 2)

    return pl.pallas_call(
        body,
        out_shape=jax.ShapeDtypeStruct((2 * m, n), x.dtype),
        in_specs=[pl.BlockSpec(memory_space=pl.ANY)],
        out_specs=pl.BlockSpec(memory_space=pl.ANY),
        scratch_shapes=[
            pltpu.SemaphoreType.DMA((K,)),
            pltpu.SemaphoreType.DMA((K,)),
            pltpu.SemaphoreType.DMA((K,)),
            pltpu.SemaphoreType.DMA((K,)),
            pltpu.VMEM((2, m // KC, n), jnp.float32),
            pltpu.SemaphoreType.DMA((2,)),
            pltpu.SemaphoreType.DMA((2,)),
        ],
        compiler_params=pltpu.CompilerParams(collective_id=0),
    )(x)
